# baseline (speedup 1.0000x reference)
import functools

import jax
import jax.numpy as jnp
from jax import lax
from jax.experimental import pallas as pl
from jax.experimental.pallas import tpu as pltpu
from jax.experimental.pallas import tpu_sc as plsc

_B = 4096
_L = 200
_N = _B * _L
_D = 64
_NW = 32
_CHUNK = 128
_PER_W = _N // _NW
_STEPS = _PER_W // _CHUNK
_VECS = _CHUNK // 16


def _body(ids_hbm, voc_hbm, len_hbm, llm_t, cod_t,
          att_o, llm_m_o, cod_m_o, llm_i_o, cod_i_o, emb_o,
          len_v, ids_v, voc_v, llm_idx_v, cod_idx_v,
          att_v, llm_m_v, cod_m_v, maskf_v,
          llm_rows, cod_rows, out_rows, sem):
    wid = lax.axis_index("s") * 2 + lax.axis_index("c")
    pltpu.sync_copy(len_hbm, len_v)

    def step(g, carry):
        base = wid * _PER_W + g * _CHUNK
        pltpu.sync_copy(ids_hbm.at[pl.ds(base, _CHUNK)], ids_v)
        pltpu.sync_copy(voc_hbm.at[pl.ds(base, _CHUNK)], voc_v)

        for v in range(_VECS):
            sl = pl.ds(v * 16, 16)
            ids = ids_v[sl]
            voc = voc_v[sl]
            flat = base + v * 16 + lax.iota(jnp.int32, 16)
            b = lax.div(flat, _L)
            pos = flat - b * _L
            ln = plsc.load_gather(len_v, [b])
            att = pos < ln
            lm = att & (voc == 0)
            cm = att & (voc == 1)
            lmi = lm.astype(jnp.int32)
            cmi = cm.astype(jnp.int32)
            llm_idx_v[sl] = ids * lmi
            cod_idx_v[sl] = ids * cmi
            att_v[sl] = att.astype(jnp.int32)
            llm_m_v[sl] = lmi
            cod_m_v[sl] = cmi
            maskf_v[sl] = att.astype(jnp.float32)

        pltpu.sync_copy(att_v, att_o.at[pl.ds(base, _CHUNK)])
        pltpu.sync_copy(llm_m_v, llm_m_o.at[pl.ds(base, _CHUNK)])
        pltpu.sync_copy(cod_m_v, cod_m_o.at[pl.ds(base, _CHUNK)])
        pltpu.sync_copy(llm_idx_v, llm_i_o.at[pl.ds(base, _CHUNK)])
        pltpu.sync_copy(cod_idx_v, cod_i_o.at[pl.ds(base, _CHUNK)])

        pltpu.async_copy(llm_t.at[llm_idx_v], llm_rows, sem).wait()
        pltpu.async_copy(cod_t.at[cod_idx_v], cod_rows, sem).wait()

        def row(r, carry2):
            m = plsc.load_gather(maskf_v, [jnp.full((16,), r, jnp.int32)])
            for c in range(_D // 16):
                cs = pl.ds(c * 16, 16)
                out_rows[r, cs] = (llm_rows[r, cs] + cod_rows[r, cs]) * m
            return carry2

        lax.fori_loop(0, _CHUNK, row, 0)
        pltpu.sync_copy(out_rows, emb_o.at[pl.ds(base, _CHUNK)])
        return carry

    lax.fori_loop(0, _STEPS, step, 0)


@jax.jit
def _run(ids_flat, voc_flat, length, llm_table, cod_table):
    i32 = jnp.int32
    out_type = (
        jax.ShapeDtypeStruct((_N,), i32),
        jax.ShapeDtypeStruct((_N,), i32),
        jax.ShapeDtypeStruct((_N,), i32),
        jax.ShapeDtypeStruct((_N,), i32),
        jax.ShapeDtypeStruct((_N,), i32),
        jax.ShapeDtypeStruct((_N, _D), jnp.float32),
    )
    scratch = [
        pltpu.VMEM((_B,), i32),
        pltpu.VMEM((_CHUNK,), i32),
        pltpu.VMEM((_CHUNK,), i32),
        pltpu.VMEM((_CHUNK,), i32),
        pltpu.VMEM((_CHUNK,), i32),
        pltpu.VMEM((_CHUNK,), i32),
        pltpu.VMEM((_CHUNK,), i32),
        pltpu.VMEM((_CHUNK,), i32),
        pltpu.VMEM((_CHUNK,), jnp.float32),
        pltpu.VMEM((_CHUNK, _D), jnp.float32),
        pltpu.VMEM((_CHUNK, _D), jnp.float32),
        pltpu.VMEM((_CHUNK, _D), jnp.float32),
        pltpu.SemaphoreType.DMA,
    ]
    mesh = plsc.VectorSubcoreMesh(core_axis_name="c", subcore_axis_name="s")
    fn = pl.kernel(_body, out_type=out_type, mesh=mesh, scratch_types=scratch,
                   compiler_params=pltpu.CompilerParams(
                       use_tc_tiling_on_sc=False, needs_layout_passes=False))
    return fn(ids_flat, voc_flat, length, llm_table, cod_table)


def kernel(input_ids, vocab_ids, length, llm_table, cod_table):
    att, lm, cm, li, ci, emb = _run(
        input_ids.reshape(_N), vocab_ids.reshape(_N), length,
        llm_table, cod_table)
    shape2 = (_B, _L)
    return (
        lm.reshape(shape2).astype(jnp.bool_),
        cm.reshape(shape2).astype(jnp.bool_),
        li.reshape(shape2),
        ci.reshape(shape2),
        att.reshape(shape2).astype(jnp.bool_),
        emb.reshape(_B, _L, _D),
    )


# speedup vs baseline: 1.0029x; 1.0029x over previous
import jax
import jax.numpy as jnp
from jax import lax
from jax.experimental import pallas as pl
from jax.experimental.pallas import tpu as pltpu
from jax.experimental.pallas import tpu_sc as plsc

_B = 4096
_L = 200
_N = _B * _L
_D = 64
_NW = 32
_CHUNK = 128
_PER_W = _N // _NW
_STEPS = _PER_W // _CHUNK
_VECS = _CHUNK // 16


def _body(ids_hbm, voc_hbm, len_hbm, llm_t, cod_t,
          att_o, llm_m_o, cod_m_o, llm_i_o, cod_i_o, emb_o,
          len_v, ids_v, voc_v, lidx_v, cidx_v,
          att_v, lm_v, cm_v, mf_v, lrow, crow,
          sem_in0, sem_in1, sem_g0, sem_g1,
          sem_os0, sem_os1, sem_e0, sem_e1):
    sem_in = (sem_in0, sem_in1)
    sem_g = (sem_g0, sem_g1)
    sem_os = (sem_os0, sem_os1)
    sem_e = (sem_e0, sem_e1)
    wid = lax.axis_index("s") * 2 + lax.axis_index("c")
    w0 = wid * _PER_W
    pltpu.sync_copy(len_hbm, len_v)

    def tok_slice(g):
        return pl.ds(w0 + g * _CHUNK, _CHUNK)

    def prefetch(g, b):
        pltpu.async_copy(ids_hbm.at[tok_slice(g)], ids_v.at[b], sem_in[b])
        pltpu.async_copy(voc_hbm.at[tok_slice(g)], voc_v.at[b], sem_in[b])

    def wait_in(g, b):
        pltpu.make_async_copy(ids_hbm.at[tok_slice(g)], ids_v.at[b],
                              sem_in[b]).wait()
        pltpu.make_async_copy(voc_hbm.at[tok_slice(g)], voc_v.at[b],
                              sem_in[b]).wait()

    def elementwise(g, b):
        base = w0 + g * _CHUNK
        for v in range(_VECS):
            sl = pl.ds(v * 16, 16)
            ids = ids_v[b, sl]
            voc = voc_v[b, sl]
            flat = base + v * 16 + lax.iota(jnp.int32, 16)
            row = lax.div(flat, _L)
            pos = flat - row * _L
            ln = plsc.load_gather(len_v, [row])
            att = pos < ln
            lm = att & (voc == 0)
            cm = att & (voc == 1)
            lmi = lm.astype(jnp.int32)
            cmi = cm.astype(jnp.int32)
            lidx_v[b, sl] = ids * lmi
            cidx_v[b, sl] = ids * cmi
            att_v[b, sl] = att.astype(jnp.int32)
            lm_v[b, sl] = lmi
            cm_v[b, sl] = cmi
            mf_v[b, sl] = att.astype(jnp.float32)

    def issue_gathers(g, b):
        pltpu.async_copy(llm_t.at[lidx_v.at[b]], lrow.at[b], sem_g[b])
        pltpu.async_copy(cod_t.at[cidx_v.at[b]], crow.at[b], sem_g[b])

    def wait_gathers(g, b):
        pltpu.make_async_copy(llm_t.at[lidx_v.at[b]], lrow.at[b],
                              sem_g[b]).wait()
        pltpu.make_async_copy(cod_t.at[cidx_v.at[b]], crow.at[b],
                              sem_g[b]).wait()

    _small = ((att_v, att_o), (lm_v, llm_m_o), (cm_v, cod_m_o),
              (lidx_v, llm_i_o), (cidx_v, cod_i_o))

    def issue_small_outs(g, b):
        for src, dst in _small:
            pltpu.async_copy(src.at[b], dst.at[tok_slice(g)], sem_os[b])

    def wait_small_outs(g, b):
        for src, dst in _small:
            pltpu.make_async_copy(src.at[b], dst.at[tok_slice(g)],
                                  sem_os[b]).wait()

    def issue_emb_out(g, b):
        pltpu.async_copy(lrow.at[b], emb_o.at[tok_slice(g)], sem_e[b])

    def wait_emb_out(g, b):
        pltpu.make_async_copy(lrow.at[b], emb_o.at[tok_slice(g)],
                              sem_e[b]).wait()

    def combine(g, b):
        def row_fn(r, carry):
            m = plsc.load_gather(mf_v.at[b], [jnp.full((16,), r, jnp.int32)])
            for c in range(_D // 16):
                cs = pl.ds(c * 16, 16)
                lrow[b, r, cs] = (lrow[b, r, cs] + crow[b, r, cs]) * m
            return carry
        lax.fori_loop(0, _CHUNK, row_fn, 0)

    def stage_a(g, b, steady):
        wait_in(g, b)
        if steady:
            wait_small_outs(g, b)
            wait_emb_out(g, b)
        elementwise(g, b)

        @pl.when(g + 2 < _STEPS)
        def _():
            prefetch(g + 2, b)

        issue_gathers(g, b)
        issue_small_outs(g, b)

    def stage_c(g, b):
        wait_gathers(g, b)
        combine(g, b)
        issue_emb_out(g, b)

    prefetch(0, 0)
    prefetch(1, 1)
    stage_a(0, 0, steady=False)
    stage_c(0, 0)
    stage_a(1, 1, steady=False)
    stage_c(1, 1)

    def pair(k, carry):
        t0 = 2 + 2 * k
        stage_a(t0, 0, steady=True)
        stage_c(t0, 0)
        stage_a(t0 + 1, 1, steady=True)
        stage_c(t0 + 1, 1)
        return carry

    lax.fori_loop(0, (_STEPS - 2) // 2, pair, 0)

    for b, g in ((0, _STEPS - 2), (1, _STEPS - 1)):
        wait_small_outs(g, b)
        wait_emb_out(g, b)


@jax.jit
def _run(ids_flat, voc_flat, length, llm_table, cod_table):
    i32 = jnp.int32
    out_type = (
        jax.ShapeDtypeStruct((_N,), i32),
        jax.ShapeDtypeStruct((_N,), i32),
        jax.ShapeDtypeStruct((_N,), i32),
        jax.ShapeDtypeStruct((_N,), i32),
        jax.ShapeDtypeStruct((_N,), i32),
        jax.ShapeDtypeStruct((_N, _D), jnp.float32),
    )
    scratch = [
        pltpu.VMEM((_B,), i32),
        pltpu.VMEM((2, _CHUNK), i32),
        pltpu.VMEM((2, _CHUNK), i32),
        pltpu.VMEM((2, _CHUNK), i32),
        pltpu.VMEM((2, _CHUNK), i32),
        pltpu.VMEM((2, _CHUNK), i32),
        pltpu.VMEM((2, _CHUNK), i32),
        pltpu.VMEM((2, _CHUNK), i32),
        pltpu.VMEM((2, _CHUNK), jnp.float32),
        pltpu.VMEM((2, _CHUNK, _D), jnp.float32),
        pltpu.VMEM((2, _CHUNK, _D), jnp.float32),
    ] + [pltpu.SemaphoreType.DMA] * 8
    mesh = plsc.VectorSubcoreMesh(core_axis_name="c", subcore_axis_name="s")
    fn = pl.kernel(_body, out_type=out_type, mesh=mesh, scratch_types=scratch,
                   compiler_params=pltpu.CompilerParams(
                       use_tc_tiling_on_sc=False, needs_layout_passes=False))
    return fn(ids_flat, voc_flat, length, llm_table, cod_table)


def kernel(input_ids, vocab_ids, length, llm_table, cod_table):
    att, lm, cm, li, ci, emb = _run(
        input_ids.reshape(_N), vocab_ids.reshape(_N), length,
        llm_table, cod_table)
    shape2 = (_B, _L)
    return (
        lm.reshape(shape2).astype(jnp.bool_),
        cm.reshape(shape2).astype(jnp.bool_),
        li.reshape(shape2),
        ci.reshape(shape2),
        att.reshape(shape2).astype(jnp.bool_),
        emb.reshape(_B, _L, _D),
    )


# speedup vs baseline: 10.7978x; 10.7668x over previous
import jax
import jax.numpy as jnp
from jax import lax
from jax.experimental import pallas as pl
from jax.experimental.pallas import tpu as pltpu
from jax.experimental.pallas import tpu_sc as plsc

_V = 100000
_B = 4096
_L = 200
_N = _B * _L
_D = 64
_NW = 32
_CHUNK = 128
_PER_W = _N // _NW
_STEPS = _PER_W // _CHUNK
_VECS = _CHUNK // 16


def _body(ids_hbm, voc_hbm, len_hbm, tab,
          att_o, llm_m_o, cod_m_o, llm_i_o, cod_i_o, emb_o,
          len_v, ids_v, voc_v, gidx_v, lidx_v, cidx_v,
          att_v, lm_v, cm_v, af_v, cf_v, row0_v, grow,
          sem_in0, sem_in1, sem_g0, sem_g1,
          sem_os0, sem_os1, sem_e0, sem_e1):
    sem_in = (sem_in0, sem_in1)
    sem_g = (sem_g0, sem_g1)
    sem_os = (sem_os0, sem_os1)
    sem_e = (sem_e0, sem_e1)
    wid = lax.axis_index("s") * 2 + lax.axis_index("c")
    w0 = wid * _PER_W
    pltpu.sync_copy(len_hbm, len_v)
    pltpu.sync_copy(tab.at[pl.ds(0, 1)], row0_v.at[pl.ds(0, 1)])
    pltpu.sync_copy(tab.at[pl.ds(_V, 1)], row0_v.at[pl.ds(1, 1)])

    def tok_slice(g):
        return pl.ds(w0 + g * _CHUNK, _CHUNK)

    def prefetch(g, b):
        pltpu.async_copy(ids_hbm.at[tok_slice(g)], ids_v.at[b], sem_in[b])
        pltpu.async_copy(voc_hbm.at[tok_slice(g)], voc_v.at[b], sem_in[b])

    def wait_in(g, b):
        pltpu.make_async_copy(ids_hbm.at[tok_slice(g)], ids_v.at[b],
                              sem_in[b]).wait()
        pltpu.make_async_copy(voc_hbm.at[tok_slice(g)], voc_v.at[b],
                              sem_in[b]).wait()

    def elementwise(g, b):
        base = w0 + g * _CHUNK
        for v in range(_VECS):
            sl = pl.ds(v * 16, 16)
            ids = ids_v[b, sl]
            voc = voc_v[b, sl]
            flat = base + v * 16 + lax.iota(jnp.int32, 16)
            row = lax.div(flat, _L)
            pos = flat - row * _L
            ln = plsc.load_gather(len_v, [row])
            att = pos < ln
            lm = att & (voc == 0)
            cm = att & (voc == 1)
            lmi = lm.astype(jnp.int32)
            cmi = cm.astype(jnp.int32)
            gidx_v[b, sl] = ids + voc * _V
            lidx_v[b, sl] = ids * lmi
            cidx_v[b, sl] = ids * cmi
            att_v[b, sl] = att.astype(jnp.int32)
            lm_v[b, sl] = lmi
            cm_v[b, sl] = cmi
            af_v[b, sl] = att.astype(jnp.float32)
            cf_v[b, sl] = cmi.astype(jnp.float32)

    def issue_gather(g, b):
        pltpu.async_copy(tab.at[gidx_v.at[b]], grow.at[b], sem_g[b])

    def wait_gather(g, b):
        pltpu.make_async_copy(tab.at[gidx_v.at[b]], grow.at[b],
                              sem_g[b]).wait()

    _small = ((att_v, att_o), (lm_v, llm_m_o), (cm_v, cod_m_o),
              (lidx_v, llm_i_o), (cidx_v, cod_i_o))

    def issue_small_outs(g, b):
        for src, dst in _small:
            pltpu.async_copy(src.at[b], dst.at[tok_slice(g)], sem_os[b])

    def wait_small_outs(g, b):
        for src, dst in _small:
            pltpu.make_async_copy(src.at[b], dst.at[tok_slice(g)],
                                  sem_os[b]).wait()

    def issue_emb_out(g, b):
        pltpu.async_copy(grow.at[b], emb_o.at[tok_slice(g)], sem_e[b])

    def wait_emb_out(g, b):
        pltpu.make_async_copy(grow.at[b], emb_o.at[tok_slice(g)],
                              sem_e[b]).wait()

    def combine(g, b):
        ncol = _D // 16
        ct0 = [row0_v[1, pl.ds(c * 16, 16)] for c in range(ncol)]
        d0 = [row0_v[0, pl.ds(c * 16, 16)] - ct0[c] for c in range(ncol)]

        def row_fn(r, carry):
            ridx = jnp.full((16,), r, jnp.int32)
            attf = plsc.load_gather(af_v.at[b], [ridx])
            cmf = plsc.load_gather(cf_v.at[b], [ridx])
            for c in range(ncol):
                cs = pl.ds(c * 16, 16)
                grow[b, r, cs] = (attf * (grow[b, r, cs] + ct0[c])
                                  + cmf * d0[c])
            return carry
        lax.fori_loop(0, _CHUNK, row_fn, 0)

    def stage_a(g, b, steady):
        wait_in(g, b)
        if steady:
            wait_small_outs(g, b)
            wait_emb_out(g, b)
        elementwise(g, b)

        @pl.when(g + 2 < _STEPS)
        def _():
            prefetch(g + 2, b)

        issue_gather(g, b)
        issue_small_outs(g, b)

    def stage_c(g, b):
        wait_gather(g, b)
        combine(g, b)
        issue_emb_out(g, b)

    prefetch(0, 0)
    prefetch(1, 1)
    stage_a(0, 0, steady=False)
    stage_c(0, 0)
    stage_a(1, 1, steady=False)
    stage_c(1, 1)

    def pair(k, carry):
        t0 = 2 + 2 * k
        stage_a(t0, 0, steady=True)
        stage_c(t0, 0)
        stage_a(t0 + 1, 1, steady=True)
        stage_c(t0 + 1, 1)
        return carry

    lax.fori_loop(0, (_STEPS - 2) // 2, pair, 0)

    for b, g in ((0, _STEPS - 2), (1, _STEPS - 1)):
        wait_small_outs(g, b)
        wait_emb_out(g, b)


@jax.jit
def _run(ids_flat, voc_flat, length, table2):
    i32 = jnp.int32
    out_type = (
        jax.ShapeDtypeStruct((_N,), i32),
        jax.ShapeDtypeStruct((_N,), i32),
        jax.ShapeDtypeStruct((_N,), i32),
        jax.ShapeDtypeStruct((_N,), i32),
        jax.ShapeDtypeStruct((_N,), i32),
        jax.ShapeDtypeStruct((_N, _D), jnp.float32),
    )
    scratch = [
        pltpu.VMEM((_B,), i32),
        pltpu.VMEM((2, _CHUNK), i32),
        pltpu.VMEM((2, _CHUNK), i32),
        pltpu.VMEM((2, _CHUNK), i32),
        pltpu.VMEM((2, _CHUNK), i32),
        pltpu.VMEM((2, _CHUNK), i32),
        pltpu.VMEM((2, _CHUNK), i32),
        pltpu.VMEM((2, _CHUNK), i32),
        pltpu.VMEM((2, _CHUNK), i32),
        pltpu.VMEM((2, _CHUNK), jnp.float32),
        pltpu.VMEM((2, _CHUNK), jnp.float32),
        pltpu.VMEM((2, _D), jnp.float32),
        pltpu.VMEM((2, _CHUNK, _D), jnp.float32),
    ] + [pltpu.SemaphoreType.DMA] * 8
    mesh = plsc.VectorSubcoreMesh(core_axis_name="c", subcore_axis_name="s")
    fn = pl.kernel(_body, out_type=out_type, mesh=mesh, scratch_types=scratch,
                   compiler_params=pltpu.CompilerParams(
                       use_tc_tiling_on_sc=False, needs_layout_passes=False))
    return fn(ids_flat, voc_flat, length, table2)


def kernel(input_ids, vocab_ids, length, llm_table, cod_table):
    table2 = jnp.concatenate([llm_table, cod_table], axis=0)
    att, lm, cm, li, ci, emb = _run(
        input_ids.reshape(_N), vocab_ids.reshape(_N), length, table2)
    shape2 = (_B, _L)
    return (
        lm.reshape(shape2).astype(jnp.bool_),
        cm.reshape(shape2).astype(jnp.bool_),
        li.reshape(shape2),
        ci.reshape(shape2),
        att.reshape(shape2).astype(jnp.bool_),
        emb.reshape(_B, _L, _D),
    )


# speedup vs baseline: 13.3451x; 1.2359x over previous
import jax
import jax.numpy as jnp
from jax import lax
from jax.experimental import pallas as pl
from jax.experimental.pallas import tpu as pltpu
from jax.experimental.pallas import tpu_sc as plsc

_V = 100000
_B = 4096
_L = 200
_N = _B * _L
_D = 64
_NW = 32
_CHUNK = 128
_PER_W = _N // _NW
_STEPS = _PER_W // _CHUNK
_VECS = _CHUNK // 16


def _body(ids_hbm, voc_hbm, len_hbm, tab,
          att_o, llm_m_o, cod_m_o, llm_i_o, cod_i_o, emb_o,
          len_v, ids_v, voc_v, gidx_v, lidx_v, cidx_v,
          att_v, lm_v, cm_v, af_v, cf_v, row0_v, grow,
          sem_in0, sem_in1, sem_in2, sem_in3,
          sem_g0, sem_g1, sem_g2, sem_g3,
          sem_os0, sem_os1, sem_os2, sem_os3,
          sem_e0, sem_e1, sem_e2, sem_e3):
    sem_in = (sem_in0, sem_in1, sem_in2, sem_in3)
    sem_g = (sem_g0, sem_g1, sem_g2, sem_g3)
    sem_os = (sem_os0, sem_os1, sem_os2, sem_os3)
    sem_e = (sem_e0, sem_e1, sem_e2, sem_e3)
    wid = lax.axis_index("s") * 2 + lax.axis_index("c")
    w0 = wid * _PER_W
    pltpu.sync_copy(len_hbm, len_v)
    pltpu.sync_copy(tab.at[pl.ds(0, 1)], row0_v.at[pl.ds(0, 1)])
    pltpu.sync_copy(tab.at[pl.ds(_V, 1)], row0_v.at[pl.ds(1, 1)])

    def tok_slice(g):
        return pl.ds(w0 + g * _CHUNK, _CHUNK)

    def prefetch(g, b):
        pltpu.async_copy(ids_hbm.at[tok_slice(g)], ids_v.at[b], sem_in[b])
        pltpu.async_copy(voc_hbm.at[tok_slice(g)], voc_v.at[b], sem_in[b])

    def wait_in(g, b):
        pltpu.make_async_copy(ids_hbm.at[tok_slice(g)], ids_v.at[b],
                              sem_in[b]).wait()
        pltpu.make_async_copy(voc_hbm.at[tok_slice(g)], voc_v.at[b],
                              sem_in[b]).wait()

    def elementwise(g, b):
        base = w0 + g * _CHUNK
        for v in range(_VECS):
            sl = pl.ds(v * 16, 16)
            ids = ids_v[b, sl]
            voc = voc_v[b, sl]
            flat = base + v * 16 + lax.iota(jnp.int32, 16)
            row = lax.div(flat, _L)
            pos = flat - row * _L
            ln = plsc.load_gather(len_v, [row])
            att = pos < ln
            lm = att & (voc == 0)
            cm = att & (voc == 1)
            lmi = lm.astype(jnp.int32)
            cmi = cm.astype(jnp.int32)
            gidx_v[b, sl] = ids + voc * _V
            lidx_v[b, sl] = ids * lmi
            cidx_v[b, sl] = ids * cmi
            att_v[b, sl] = att.astype(jnp.int32)
            lm_v[b, sl] = lmi
            cm_v[b, sl] = cmi
            af_v[b, sl] = att.astype(jnp.float32)
            cf_v[b, sl] = cmi.astype(jnp.float32)

    def issue_gather(g, b):
        pltpu.async_copy(tab.at[gidx_v.at[b]], grow.at[b], sem_g[b])

    def wait_gather(g, b):
        pltpu.make_async_copy(tab.at[gidx_v.at[b]], grow.at[b],
                              sem_g[b]).wait()

    _small = ((att_v, att_o), (lm_v, llm_m_o), (cm_v, cod_m_o),
              (lidx_v, llm_i_o), (cidx_v, cod_i_o))

    def issue_small_outs(g, b):
        for src, dst in _small:
            pltpu.async_copy(src.at[b], dst.at[tok_slice(g)], sem_os[b])

    def wait_small_outs(g, b):
        for src, dst in _small:
            pltpu.make_async_copy(src.at[b], dst.at[tok_slice(g)],
                                  sem_os[b]).wait()

    def issue_emb_out(g, b):
        pltpu.async_copy(grow.at[b], emb_o.at[tok_slice(g)], sem_e[b])

    def wait_emb_out(g, b):
        pltpu.make_async_copy(grow.at[b], emb_o.at[tok_slice(g)],
                              sem_e[b]).wait()

    def combine(g, b):
        ncol = _D // 16
        ct0 = [row0_v[1, pl.ds(c * 16, 16)] for c in range(ncol)]
        d0 = [row0_v[0, pl.ds(c * 16, 16)] - ct0[c] for c in range(ncol)]

        def row_fn(r, carry):
            ridx = jnp.full((16,), r, jnp.int32)
            attf = plsc.load_gather(af_v.at[b], [ridx])
            cmf = plsc.load_gather(cf_v.at[b], [ridx])
            for c in range(ncol):
                cs = pl.ds(c * 16, 16)
                grow[b, r, cs] = (attf * (grow[b, r, cs] + ct0[c])
                                  + cmf * d0[c])
            return carry
        lax.fori_loop(0, _CHUNK, row_fn, 0)

    def stage_a(g, b, steady):
        wait_in(g, b)
        if steady:
            wait_small_outs(g, b)
            wait_emb_out(g, b)
        elementwise(g, b)

        @pl.when(g + 4 < _STEPS)
        def _():
            prefetch(g + 4, b)

        issue_gather(g, b)
        issue_small_outs(g, b)

    def stage_c(g, b):
        wait_gather(g, b)
        combine(g, b)
        issue_emb_out(g, b)

    for b in range(4):
        prefetch(b, b)
    stage_a(0, 0, steady=False)
    stage_a(1, 1, steady=False)
    stage_a(2, 2, steady=False)
    stage_a(3, 3, steady=False)
    stage_c(0, 0)
    stage_c(1, 1)

    def quad(k, carry):
        g0 = 4 + 4 * k
        stage_a(g0, 0, steady=True)
        stage_c(g0 - 2, 2)
        stage_a(g0 + 1, 1, steady=True)
        stage_c(g0 - 1, 3)
        stage_a(g0 + 2, 2, steady=True)
        stage_c(g0, 0)
        stage_a(g0 + 3, 3, steady=True)
        stage_c(g0 + 1, 1)
        return carry

    lax.fori_loop(0, (_STEPS - 4) // 4, quad, 0)

    stage_c(_STEPS - 2, 2)
    stage_c(_STEPS - 1, 3)
    for b, g in ((0, _STEPS - 4), (1, _STEPS - 3),
                 (2, _STEPS - 2), (3, _STEPS - 1)):
        wait_small_outs(g, b)
        wait_emb_out(g, b)


@jax.jit
def _run(ids_flat, voc_flat, length, table2):
    i32 = jnp.int32
    out_type = (
        jax.ShapeDtypeStruct((_N,), i32),
        jax.ShapeDtypeStruct((_N,), i32),
        jax.ShapeDtypeStruct((_N,), i32),
        jax.ShapeDtypeStruct((_N,), i32),
        jax.ShapeDtypeStruct((_N,), i32),
        jax.ShapeDtypeStruct((_N, _D), jnp.float32),
    )
    scratch = [
        pltpu.VMEM((_B,), i32),
        pltpu.VMEM((4, _CHUNK), i32),
        pltpu.VMEM((4, _CHUNK), i32),
        pltpu.VMEM((4, _CHUNK), i32),
        pltpu.VMEM((4, _CHUNK), i32),
        pltpu.VMEM((4, _CHUNK), i32),
        pltpu.VMEM((4, _CHUNK), i32),
        pltpu.VMEM((4, _CHUNK), i32),
        pltpu.VMEM((4, _CHUNK), i32),
        pltpu.VMEM((4, _CHUNK), jnp.float32),
        pltpu.VMEM((4, _CHUNK), jnp.float32),
        pltpu.VMEM((2, _D), jnp.float32),
        pltpu.VMEM((4, _CHUNK, _D), jnp.float32),
    ] + [pltpu.SemaphoreType.DMA] * 16
    mesh = plsc.VectorSubcoreMesh(core_axis_name="c", subcore_axis_name="s")
    fn = pl.kernel(_body, out_type=out_type, mesh=mesh, scratch_types=scratch,
                   compiler_params=pltpu.CompilerParams(
                       use_tc_tiling_on_sc=False, needs_layout_passes=False))
    return fn(ids_flat, voc_flat, length, table2)


def kernel(input_ids, vocab_ids, length, llm_table, cod_table):
    table2 = jnp.concatenate([llm_table, cod_table], axis=0)
    att, lm, cm, li, ci, emb = _run(
        input_ids.reshape(_N), vocab_ids.reshape(_N), length, table2)
    shape2 = (_B, _L)
    return (
        lm.reshape(shape2).astype(jnp.bool_),
        cm.reshape(shape2).astype(jnp.bool_),
        li.reshape(shape2),
        ci.reshape(shape2),
        att.reshape(shape2).astype(jnp.bool_),
        emb.reshape(_B, _L, _D),
    )
